# baseline (speedup 1.0000x reference)
import functools

import jax
import jax.numpy as jnp
from jax import lax
from jax.experimental import pallas as pl
from jax.experimental.pallas import tpu as pltpu
from jax.experimental.pallas import tpu_sc as plsc

N_ENT = 1000000
DIM = 32
BATCH = 16384
INT_TEMP = 0.01
VOL_TEMP = 1.0

NC, NS = 2, 16
NW = NC * NS
ROWS = 2 * BATCH
R_PER_W = ROWS // NW
CH = 128
NCH = R_PER_W // CH

@functools.lru_cache(maxsize=None)
def _get_gather():
    mesh = plsc.VectorSubcoreMesh(
        core_axis_name="c", subcore_axis_name="s", num_cores=NC, num_subcores=NS
    )

    @functools.partial(
        pl.kernel,
        mesh=mesh,
        out_type=jax.ShapeDtypeStruct((ROWS, 2, DIM), jnp.float32),
        scratch_types=[
            pltpu.VMEM((NCH, CH), jnp.int32),
            pltpu.VMEM((R_PER_W, 2, DIM), jnp.float32),
            pltpu.SemaphoreType.DMA,
        ],
        compiler_params=pltpu.CompilerParams(use_tc_tiling_on_sc=False),
    )
    def _gather(table_hbm, idx_hbm, out_hbm, idx_v, rows_v, sem):
        wid = lax.axis_index("s") * NC + lax.axis_index("c")
        base = wid * R_PER_W
        pltpu.sync_copy(idx_hbm.at[wid], idx_v)
        copies = []
        for j in range(NCH):
            copies.append(
                pltpu.async_copy(
                    table_hbm.at[idx_v.at[j]],
                    rows_v.at[pl.ds(j * CH, CH)],
                    sem,
                )
            )
        for c in copies:
            c.wait()
        pltpu.sync_copy(rows_v, out_hbm.at[pl.ds(base, R_PER_W)])

    return _gather


_TC_ROWS = 2048


def _energy_body(g_ref, out_ref):
    g = g_ref[...]
    z0 = g[:, 0:DIM]
    nz0 = g[:, DIM:2 * DIM]
    z1 = g[:, 2 * DIM:3 * DIM]
    nz1 = g[:, 3 * DIM:4 * DIM]

    def gumbel_lse(a, b):
        m = jnp.maximum(a, b)
        lo = jnp.minimum(a, b)
        return m + INT_TEMP * jnp.log1p(jnp.exp((lo - m) / INT_TEMP))

    inter_z = gumbel_lse(z0, z1)
    inter_nz = gumbel_lse(nz0, nz1)
    side0 = -(inter_z + inter_nz)
    side1 = -(z1 + nz1)

    def log_vol_terms(s):
        s = s / VOL_TEMP
        sp = jnp.maximum(s, 0.0) + jnp.log1p(jnp.exp(-jnp.abs(s)))
        return jnp.log(VOL_TEMP * sp + 1e-23)

    d = jnp.sum(log_vol_terms(side0) - log_vol_terms(side1), axis=1)
    out_ref[...] = d.reshape(_TC_ROWS // 128, 128)


_energy = pl.pallas_call(
    _energy_body,
    grid=(BATCH // _TC_ROWS,),
    in_specs=[pl.BlockSpec((_TC_ROWS, 4 * DIM), lambda i: (i, 0))],
    out_specs=pl.BlockSpec((_TC_ROWS // 128, 128), lambda i: (i, 0)),
    out_shape=jax.ShapeDtypeStruct((BATCH // 128, 128), jnp.float32),
)


def kernel(idxs, boxes):
    idx3 = idxs.reshape(NW, NCH, CH)
    g = _get_gather()(boxes, idx3)
    out = _energy(g.reshape(BATCH, 4 * DIM))
    return out.reshape(BATCH)


# speedup vs baseline: 3.6193x; 3.6193x over previous
import functools

import jax
import jax.numpy as jnp
from jax import lax
from jax.experimental import pallas as pl
from jax.experimental.pallas import tpu as pltpu
from jax.experimental.pallas import tpu_sc as plsc

N_ENT = 1000000
DIM = 32
BATCH = 16384
INT_TEMP = 0.01

NC, NS = 2, 16
NW = NC * NS
EPW = BATCH // NW
RPW = 2 * EPW
CH = 128
NCH = RPW // CH

_C1, _C2, _C3, _C4 = -0.49944055, 0.32098896, -0.18417763, 0.05617695
_LN2 = 0.6931471805599453


def _log1p01(u):
    return u * (1.0 + u * (_C1 + u * (_C2 + u * (_C3 + u * _C4))))


def _swlog(x):
    bits = plsc.bitcast(x, jnp.int32)
    e = (bits >> 23) - 127
    m = plsc.bitcast((bits & 0x7FFFFF) | 0x3F800000, jnp.float32)
    w = (m - 1.0) / (m + 1.0)
    w2 = w * w
    return _LN2 * e.astype(jnp.float32) + w * (
        2.0 + w2 * (2.0 / 3.0 + w2 * (0.4 + w2 * (2.0 / 7.0))))


def _lse_pair(a, b):
    m = jnp.maximum(a, b)
    lo = jnp.minimum(a, b)
    u = jnp.exp((lo - m) * (1.0 / INT_TEMP))
    return m + INT_TEMP * _log1p01(u)


def _log_vol_term(s):
    e = jnp.exp(jnp.minimum(s, -s))
    sp = jnp.maximum(s, 0.0) + _log1p01(e) + 1e-23
    return _swlog(sp)


@functools.lru_cache(maxsize=None)
def _get_sc():
    mesh = plsc.VectorSubcoreMesh(
        core_axis_name="c", subcore_axis_name="s", num_cores=NC, num_subcores=NS
    )

    @functools.partial(
        pl.kernel,
        mesh=mesh,
        out_type=jax.ShapeDtypeStruct((BATCH,), jnp.float32),
        scratch_types=[
            pltpu.VMEM((NCH, CH), jnp.int32),
            pltpu.VMEM((RPW, 2 * DIM), jnp.float32),
            pltpu.VMEM((EPW,), jnp.float32),
            pltpu.SemaphoreType.DMA,
        ],
        compiler_params=pltpu.CompilerParams(
            use_tc_tiling_on_sc=False, needs_layout_passes=False
        ),
    )
    def sc_energy(table, idx3, out, idx_v, rows_v, ene, sem):
        cid = lax.axis_index("c")
        sid = lax.axis_index("s")
        wid = sid * NC + cid
        pltpu.sync_copy(idx3.at[wid], idx_v)
        copies = []
        for j in range(NCH):
            copies.append(pltpu.async_copy(
                table.at[idx_v.at[j]], rows_v.at[pl.ds(j * CH, CH)], sem))
        for cp in copies:
            cp.wait()

        lanes = lax.iota(jnp.int32, 16)

        def _elem(i, vacc):
            r0 = 2 * i
            r1 = 2 * i + 1
            a0 = [rows_v[r0, pl.ds(16 * q, 16)] for q in range(4)]
            a1 = [rows_v[r1, pl.ds(16 * q, 16)] for q in range(4)]
            it = [_lse_pair(a0[q], a1[q]) for q in range(4)]
            s0a = -(it[0] + it[2])
            s0b = -(it[1] + it[3])
            s1a = -(a1[0] + a1[2])
            s1b = -(a1[1] + a1[3])
            t = (_log_vol_term(s0a) + _log_vol_term(s0b)
                 - _log_vol_term(s1a) - _log_vol_term(s1b))
            vacc = jnp.where(lanes == (i & 15), jnp.sum(t), vacc)

            @pl.when((i & 15) == 15)
            def _():
                ene[pl.ds(i - 15, 16)] = vacc

            return vacc

        lax.fori_loop(0, EPW, _elem, jnp.zeros((16,), jnp.float32))
        pltpu.sync_copy(ene, out.at[pl.ds(EPW * wid, EPW)])

    return sc_energy


def kernel(idxs, boxes):
    table = boxes.reshape(N_ENT, 2 * DIM)
    idx3 = idxs.reshape(NW, NCH, CH)
    return _get_sc()(table, idx3)


# speedup vs baseline: 7.1908x; 1.9868x over previous
import functools

import jax
import jax.numpy as jnp
from jax import lax
from jax.experimental import pallas as pl
from jax.experimental.pallas import tpu as pltpu
from jax.experimental.pallas import tpu_sc as plsc

N_ENT = 1000000
DIM = 32
BATCH = 16384
INT_TEMP = 0.01

_KT = 21
_GT = 94
TPAD = _GT * 512 * _KT

NC, NS = 2, 16
NW = NC * NS
EPW = BATCH // NW
RPW = 2 * EPW
CH = 128
NCH = RPW // CH
EPH = EPW // 2

_C1, _C2, _C3, _C4 = -0.49944055, 0.32098896, -0.18417763, 0.05617695
_LN2 = 0.6931471805599453


def _log1p01(u):
    return u * (1.0 + u * (_C1 + u * (_C2 + u * (_C3 + u * _C4))))


def _swlog(x):
    bits = plsc.bitcast(x, jnp.int32)
    e = (bits >> 23) - 127
    m = plsc.bitcast((bits & 0x7FFFFF) | 0x3F800000, jnp.float32)
    w = (m - 1.0) / (m + 1.0)
    w2 = w * w
    return _LN2 * e.astype(jnp.float32) + w * (
        2.0 + w2 * (2.0 / 3.0 + w2 * (0.4 + w2 * (2.0 / 7.0))))


def _lse_pair(a, b):
    m = jnp.maximum(a, b)
    lo = jnp.minimum(a, b)
    u = jnp.exp((lo - m) * (1.0 / INT_TEMP))
    return m + INT_TEMP * _log1p01(u)


def _log_vol_term(s):
    e = jnp.exp(jnp.minimum(s, -s))
    sp = jnp.maximum(s, 0.0) + _log1p01(e) + 1e-23
    return _swlog(sp)


def _relayout_body(a_ref, o_ref):
    for k in range(_KT):
        o_ref[pl.ds(512 * k, 512), 0:64] = a_ref[:, pl.ds(512 * k, 512)].T


_relayout = pl.pallas_call(
    _relayout_body,
    grid=(_GT,),
    in_specs=[pl.BlockSpec((2 * DIM, 512 * _KT), lambda i: (0, i))],
    out_specs=pl.BlockSpec((512 * _KT, 128), lambda i: (i, 0)),
    out_shape=jax.ShapeDtypeStruct((TPAD, 128), jnp.float32),
)


@functools.lru_cache(maxsize=None)
def _get_sc():
    mesh = plsc.VectorSubcoreMesh(
        core_axis_name="c", subcore_axis_name="s", num_cores=NC, num_subcores=NS
    )

    @functools.partial(
        pl.kernel,
        mesh=mesh,
        out_type=jax.ShapeDtypeStruct((BATCH,), jnp.float32),
        scratch_types=[
            pltpu.VMEM((NCH, CH), jnp.int32),
            pltpu.VMEM((RPW // 2, 128), jnp.float32),
            pltpu.VMEM((EPW,), jnp.float32),
            pltpu.SemaphoreType.DMA,
        ],
        compiler_params=pltpu.CompilerParams(needs_layout_passes=False),
    )
    def sc_energy(table, idx3, out, idx_v, rows_v, ene, sem):
        cid = lax.axis_index("c")
        sid = lax.axis_index("s")
        wid = sid * NC + cid
        pltpu.sync_copy(idx3.at[wid], idx_v)

        lanes = lax.iota(jnp.int32, 16)
        for h in range(2):
            copies = []
            for j in range(NCH // 2):
                copies.append(pltpu.async_copy(
                    table.at[idx_v.at[(NCH // 2) * h + j]],
                    rows_v.at[pl.ds(CH * j, CH)], sem))
            for cp in copies:
                cp.wait()

            def _elem(i, vacc):
                r0 = 2 * i
                r1 = 2 * i + 1
                a0 = [rows_v[r0, pl.ds(16 * q, 16)] for q in range(4)]
                a1 = [rows_v[r1, pl.ds(16 * q, 16)] for q in range(4)]
                it = [_lse_pair(a0[q], a1[q]) for q in range(4)]
                s0a = -(it[0] + it[2])
                s0b = -(it[1] + it[3])
                s1a = -(a1[0] + a1[2])
                s1b = -(a1[1] + a1[3])
                t = (_log_vol_term(s0a) + _log_vol_term(s0b)
                     - _log_vol_term(s1a) - _log_vol_term(s1b))
                vacc = jnp.where(lanes == (i & 15), jnp.sum(t), vacc)

                @pl.when((i & 15) == 15)
                def _():
                    ene[pl.ds(EPH * h + i - 15, 16)] = vacc

                return vacc

            lax.fori_loop(0, EPH, _elem, jnp.zeros((16,), jnp.float32))

        pltpu.sync_copy(ene, out.at[pl.ds(EPW * wid, EPW)])

    return sc_energy


def kernel(idxs, boxes):
    t4 = boxes.transpose(1, 2, 0).reshape(2 * DIM, N_ENT)
    big = _relayout(t4)
    idx3 = idxs.reshape(NW, NCH, CH)
    return _get_sc()(big, idx3)
